# Initial kernel scaffold; baseline (speedup 1.0000x reference)
import jax
import jax.numpy as jnp
from jax.experimental import pallas as pl


def kernel(x, y):
    raise NotImplementedError("write your pallas kernel here")


# speedup vs baseline: 1.8343x; 1.8343x over previous
import functools

import jax
import jax.numpy as jnp
from jax import lax
from jax.experimental import pallas as pl
from jax.experimental.pallas import tpu as pltpu
from jax.experimental.pallas import tpu_sc as plsc

N = 1024 * 200
NC, NS = 2, 16
NW = NC * NS
RW = N // NW
SEL = (3, 1, 2)
GROUPS = RW // 16


def _build():
    mesh = plsc.VectorSubcoreMesh(core_axis_name="c", subcore_axis_name="s")

    @functools.partial(
        pl.kernel,
        mesh=mesh,
        out_type=jax.ShapeDtypeStruct((3 * N,), jnp.float32),
        scratch_types=[
            pltpu.VMEM((RW,), jnp.int32),
            pltpu.VMEM((RW,), jnp.int32),
            pltpu.VMEM((RW,), jnp.int32),
            pltpu.VMEM((RW,), jnp.float32),
            pltpu.VMEM((RW,), jnp.float32),
            pltpu.VMEM((RW,), jnp.float32),
            pltpu.VMEM((RW,), jnp.float32),
            pltpu.VMEM((RW,), jnp.float32),
            pltpu.VMEM((RW,), jnp.float32),
            pltpu.VMEM((RW,), jnp.float32),
            pltpu.VMEM((RW,), jnp.float32),
            pltpu.VMEM((RW,), jnp.float32),
            pltpu.SemaphoreType.DMA,
            pltpu.SemaphoreType.DMA,
            pltpu.SemaphoreType.DMA,
            pltpu.SemaphoreType.DMA,
            pltpu.SemaphoreType.DMA,
            pltpu.SemaphoreType.DMA,
            pltpu.SemaphoreType.DMA,
        ],
    )
    def run(x_hbm, yt_hbm, out_hbm,
            ix0, ix1, ix2, xc0, xc1, xc2, yc0, yc1, yc2, oc0, oc1, oc2,
            sx0, sx1, sx2, sy0, sy1, sy2, so):
        wid = lax.axis_index("s") * NC + lax.axis_index("c")
        row0 = wid * RW
        ixs = (ix0, ix1, ix2)
        xcs = (xc0, xc1, xc2)
        ycs = (yc0, yc1, yc2)
        ocs = (oc0, oc1, oc2)
        sxs = (sx0, sx1, sx2)
        sys_ = (sy0, sy1, sy2)

        iota = lax.iota(jnp.int32, 16)
        step = lax.mul(iota, jnp.full((16,), 25600, dtype=jnp.int32))

        ycopies = []
        for j in range(3):
            cy = pltpu.make_async_copy(
                yt_hbm.at[pl.ds(j * N + row0, RW)], ycs[j], sys_[j])
            cy.start()
            ycopies.append(cy)

        def idx_body(g, carry):
            p = row0 + g * 16
            b = p // 1024
            a0 = p % 1024
            base = a0 * 25600 + b * 128
            for j in range(3):
                bv = jnp.full((16,), base + SEL[j], dtype=jnp.int32)
                ixs[j][pl.ds(g * 16, 16)] = lax.add(bv, step)
            return carry

        lax.fori_loop(0, GROUPS, idx_body, 0)

        gathers = []
        ocopies = []
        for j in range(3):
            cg = pltpu.make_async_copy(x_hbm.at[ixs[j]], xcs[j], sxs[j])
            cg.start()
            gathers.append(cg)

        for j in range(3):
            gathers[j].wait()
            ycopies[j].wait()

            def add_body(k, carry, _j=j):
                ocs[_j][pl.ds(k * 16, 16)] = (
                    xcs[_j][pl.ds(k * 16, 16)] + ycs[_j][pl.ds(k * 16, 16)])
                return carry

            lax.fori_loop(0, GROUPS, add_body, 0)
            co = pltpu.make_async_copy(
                ocs[j], out_hbm.at[pl.ds(j * N + row0, RW)], so)
            co.start()
            ocopies.append(co)
        for co in ocopies:
            co.wait()

    return run


_RUN = _build()


@jax.jit
def kernel(x, y):
    xr = x.reshape(N * 128)
    yt = y.transpose(2, 1, 0).reshape(3 * N)
    out_t = _RUN(xr, yt)
    return out_t.reshape(3, 200, 1024).transpose(2, 1, 0)


# speedup vs baseline: 1.9735x; 1.0759x over previous
import functools

import jax
import jax.numpy as jnp
from jax import lax
from jax.experimental import pallas as pl
from jax.experimental.pallas import tpu as pltpu
from jax.experimental.pallas import tpu_sc as plsc

N = 1024 * 200
NC, NS = 2, 16
NW = NC * NS
RW = N // NW
SEL = (3, 1, 2)
GROUPS = RW // 16


def _build():
    mesh = plsc.VectorSubcoreMesh(core_axis_name="c", subcore_axis_name="s")

    @functools.partial(
        pl.kernel,
        mesh=mesh,
        out_type=jax.ShapeDtypeStruct((3 * N,), jnp.float32),
        scratch_types=[
            pltpu.VMEM((RW,), jnp.int32),
            pltpu.VMEM((RW,), jnp.int32),
            pltpu.VMEM((RW,), jnp.int32),
            pltpu.VMEM((RW,), jnp.float32),
            pltpu.VMEM((RW,), jnp.float32),
            pltpu.VMEM((RW,), jnp.float32),
            pltpu.VMEM((RW,), jnp.float32),
            pltpu.VMEM((RW,), jnp.float32),
            pltpu.VMEM((RW,), jnp.float32),
            pltpu.VMEM((RW,), jnp.float32),
            pltpu.VMEM((RW,), jnp.float32),
            pltpu.VMEM((RW,), jnp.float32),
            pltpu.SemaphoreType.DMA,
            pltpu.SemaphoreType.DMA,
            pltpu.SemaphoreType.DMA,
            pltpu.SemaphoreType.DMA,
            pltpu.SemaphoreType.DMA,
            pltpu.SemaphoreType.DMA,
            pltpu.SemaphoreType.DMA,
        ],
    )
    def run(x_hbm, yt_hbm, out_hbm,
            ix0, ix1, ix2, xc0, xc1, xc2, yc0, yc1, yc2, oc0, oc1, oc2,
            sx0, sx1, sx2, sy0, sy1, sy2, so):
        wid = lax.axis_index("s") * NC + lax.axis_index("c")
        row0 = wid * RW
        ixs = (ix0, ix1, ix2)
        xcs = (xc0, xc1, xc2)
        ycs = (yc0, yc1, yc2)
        ocs = (oc0, oc1, oc2)
        sxs = (sx0, sx1, sx2)
        sys_ = (sy0, sy1, sy2)

        iota = lax.iota(jnp.int32, 16)
        step = lax.mul(iota, jnp.full((16,), 25600, dtype=jnp.int32))

        ycopies = []
        for j in range(3):
            cy = pltpu.make_async_copy(
                yt_hbm.at[pl.ds(j * N + row0, RW)], ycs[j], sys_[j])
            cy.start()
            ycopies.append(cy)

        def idx_body(g4, carry):
            for u in range(4):
                g = g4 * 4 + u
                p = row0 + g * 16
                b = p // 1024
                a0 = p % 1024
                base = a0 * 25600 + b * 128
                for j in range(3):
                    bv = jnp.full((16,), base + SEL[j], dtype=jnp.int32)
                    ixs[j][pl.ds(g * 16, 16)] = lax.add(bv, step)
            return carry

        lax.fori_loop(0, GROUPS // 4, idx_body, 0)

        gathers = []
        ocopies = []
        for j in range(3):
            cg = pltpu.make_async_copy(x_hbm.at[ixs[j]], xcs[j], sxs[j])
            cg.start()
            gathers.append(cg)

        for j in range(3):
            gathers[j].wait()
            ycopies[j].wait()

            def add_body(k4, carry, _j=j):
                for u in range(4):
                    k = k4 * 4 + u
                    ocs[_j][pl.ds(k * 16, 16)] = (
                        xcs[_j][pl.ds(k * 16, 16)]
                        + ycs[_j][pl.ds(k * 16, 16)])
                return carry

            lax.fori_loop(0, GROUPS // 4, add_body, 0)
            co = pltpu.make_async_copy(
                ocs[j], out_hbm.at[pl.ds(j * N + row0, RW)], so)
            co.start()
            ocopies.append(co)
        for co in ocopies:
            co.wait()

    return run


_RUN = _build()


@jax.jit
def kernel(x, y):
    xr = x.reshape(N * 128)
    yt = y.transpose(2, 1, 0).reshape(3 * N)
    out_t = _RUN(xr, yt)
    return out_t.reshape(3, 200, 1024).transpose(2, 1, 0)
